# baseline (speedup 1.0000x reference)
import functools

import numpy as np
import jax
import jax.numpy as jnp
from jax import lax
from jax.experimental import pallas as pl
from jax.experimental.pallas import tpu as pltpu
from jax.experimental.pallas import tpu_sc as plsc

D_MODEL = 64
SEQ_LEN = 200
BATCH = 1024
V_ROWS = 1000000
SCALE = np.float32(np.sqrt(D_MODEL))

FULL_BLOCKS = V_ROWS // 128
TAIL_ROWS = V_ROWS - FULL_BLOCKS * 128
Z_ROWS = FULL_BLOCKS * 64 + TAIL_ROWS // 2
Z_PAD = 64
NBUF_T = 6
NBUF_G = 4

_SPLIT_A = 128
_SPLIT_B = SEQ_LEN - _SPLIT_A


def _positional_encoding(length, depth):
    half = depth / 2
    positions = np.arange(length)[:, np.newaxis]
    depths = np.arange(half)[np.newaxis, :] / half
    angle_rates = 1 / 10000 ** depths
    angle_rads = positions * angle_rates
    pe = np.concatenate([np.sin(angle_rads), np.cos(angle_rads)], axis=-1)
    return pe.astype(np.float32)


_PE_NP = _positional_encoding(SEQ_LEN, D_MODEL)


@functools.cache
def _build_transpose():
    info = plsc.get_sparse_core_info()
    nc, ns = info.num_cores, info.num_subcores
    nw = nc * ns
    base_n = FULL_BLOCKS // nw
    extra = FULL_BLOCKS - base_n * nw
    slots = base_n + 2
    outer = slots // NBUF_T
    assert slots % NBUF_T == 0 and slots >= base_n + 1
    mesh = plsc.VectorSubcoreMesh(core_axis_name="c", subcore_axis_name="s")

    @functools.partial(
        pl.kernel,
        mesh=mesh,
        out_type=jax.ShapeDtypeStruct((Z_ROWS + Z_PAD, 128), jnp.float32),
        scratch_types=[
            [pltpu.VMEM((64, 128), jnp.float32)] * NBUF_T,
            [pltpu.VMEM((64, 128), jnp.float32)] * NBUF_T,
            pltpu.VMEM((64, 64), jnp.float32),
            pltpu.VMEM((32, 128), jnp.float32),
            [pltpu.SemaphoreType.DMA] * NBUF_T,
            [pltpu.SemaphoreType.DMA] * NBUF_T,
        ],
        compiler_params=pltpu.CompilerParams(needs_layout_passes=False),
    )
    def _transpose(tbl_t, z, in_bufs, out_bufs, tail_in, tail_out,
                   isems, osems):
        wid = lax.axis_index("s") * nc + lax.axis_index("c")
        n_w = base_n + jnp.where(wid < extra, 1, 0)
        start = base_n * wid + jnp.minimum(wid, extra)

        iota = lax.iota(jnp.int32, 16)
        rows_t = [iota + 16 * t for t in range(4)]
        diag = [(iota + k) & 15 for k in range(16)]
        dshr = [d >> 1 for d in diag]
        colv8 = [[rows_t[cb] + ((diag[k & 1] & 1) << 6) for cb in range(4)]
                 for k in range(2)]

        def src_col(blk):
            g = jnp.where(blk < n_w, start + blk, 0)
            return g * 128

        def dst_row(blk):
            return jnp.where(blk < n_w, (start + blk) * 64, Z_ROWS)

        def fire_in(blk, b):
            pltpu.async_copy(
                tbl_t.at[:, pl.ds(src_col(blk), 128)], in_bufs[b], isems[b])

        def fire_out(blk, b):
            pltpu.async_copy(
                out_bufs[b], z.at[pl.ds(dst_row(blk), 64)], osems[b])

        def transpose_block(src, dst):
            def body(jb, carry):
                j0 = jb * 16
                jhalf = jb * 8
                for k in range(16):
                    srccol = diag[k] + j0
                    rowv = dshr[k] + jhalf
                    for cb in range(4):
                        v = plsc.load_gather(src, [rows_t[cb], srccol])
                        plsc.store_scatter(dst, [rowv, colv8[k & 1][cb]], v)
                return carry
            lax.fori_loop(0, 8, body, 0, unroll=4)

        for b in range(NBUF_T):
            fire_in(b, b)

        def outer_body(k, carry):
            for b in range(NBUF_T):
                blk = k * NBUF_T + b
                pltpu.make_async_copy(
                    tbl_t.at[:, pl.ds(0, 128)], in_bufs[b], isems[b]).wait()

                @pl.when(k > 0)
                def _():
                    pltpu.make_async_copy(
                        out_bufs[b], z.at[pl.ds(0, 64)], osems[b]).wait()

                transpose_block(in_bufs[b], out_bufs[b])
                fire_out(blk, b)

                @pl.when(k < outer - 1)
                def _():
                    fire_in(blk + NBUF_T, b)
            return carry

        lax.fori_loop(0, outer, outer_body, 0)
        for b in range(NBUF_T):
            pltpu.make_async_copy(
                out_bufs[b], z.at[pl.ds(0, 64)], osems[b]).wait()

        @pl.when(wid == nw - 1)
        def _():
            pltpu.sync_copy(tbl_t.at[:, pl.ds(FULL_BLOCKS * 128, TAIL_ROWS)],
                            tail_in)

            def tail_body(p, carry):
                c0 = jnp.full((16,), 2 * p, jnp.int32)
                c1 = c0 + 1
                for t in range(4):
                    v = plsc.load_gather(tail_in, [rows_t[t], c0])
                    tail_out[p, pl.ds(16 * t, 16)] = v
                for t in range(4):
                    v = plsc.load_gather(tail_in, [rows_t[t], c1])
                    tail_out[p, pl.ds(64 + 16 * t, 16)] = v
                return carry
            lax.fori_loop(0, TAIL_ROWS // 2, tail_body, 0)
            pltpu.sync_copy(tail_out,
                            z.at[pl.ds(FULL_BLOCKS * 64, TAIL_ROWS // 2)])

    return _transpose


@functools.cache
def _build_emb_lookup():
    info = plsc.get_sparse_core_info()
    nc, ns = info.num_cores, info.num_subcores
    nw = nc * ns
    seq_per_w = BATCH // nw
    n_idx = seq_per_w * SEQ_LEN
    mesh = plsc.VectorSubcoreMesh(core_axis_name="c", subcore_axis_name="s")

    @functools.partial(
        pl.kernel,
        mesh=mesh,
        out_type=jax.ShapeDtypeStruct((BATCH, SEQ_LEN, D_MODEL), jnp.float32),
        scratch_types=[
            pltpu.VMEM((n_idx,), jnp.int32),
            pltpu.VMEM((SEQ_LEN, D_MODEL), jnp.float32),
            [pltpu.VMEM((SEQ_LEN, D_MODEL), jnp.float32)] * NBUF_G,
            [pltpu.SemaphoreType.DMA] * NBUF_G,
            [pltpu.SemaphoreType.DMA] * NBUF_G,
        ],
        compiler_params=pltpu.CompilerParams(use_tc_tiling_on_sc=False),
    )
    def _emb_lookup(idx_hbm, z_hbm, pe_hbm, out_hbm,
                    idx_v, pe_v, bufs, gsems, osems):
        wid = lax.axis_index("s") * nc + lax.axis_index("c")
        w_base = wid * seq_per_w

        pltpu.sync_copy(pe_hbm, pe_v)
        pltpu.sync_copy(idx_hbm.at[pl.ds(w_base * SEQ_LEN, n_idx)], idx_v)

        def fire_gather(s, b):
            base = s * SEQ_LEN
            c1 = pltpu.async_copy(
                z_hbm.at[idx_v.at[pl.ds(base, _SPLIT_A)]],
                bufs[b].at[pl.ds(0, _SPLIT_A)], gsems[b])
            c2 = pltpu.async_copy(
                z_hbm.at[idx_v.at[pl.ds(base + _SPLIT_A, _SPLIT_B)]],
                bufs[b].at[pl.ds(_SPLIT_A, _SPLIT_B)], gsems[b])
            return c1, c2

        pending_g = {}
        pending_o = {}
        for s in range(NBUF_G - 1):
            pending_g[s] = fire_gather(s, s)

        for s in range(seq_per_w):
            b = s % NBUF_G
            c1, c2 = pending_g.pop(s)
            c1.wait()
            c2.wait()
            buf = bufs[b]

            def row_body(r, c, buf=buf):
                for j in range(D_MODEL // 16):
                    sl = pl.ds(j * 16, 16)
                    buf[r, sl] = buf[r, sl] * SCALE + pe_v[r, sl]
                return c

            lax.fori_loop(0, SEQ_LEN, row_body, 0, unroll=2)

            pending_o[s] = pltpu.async_copy(
                buf, out_hbm.at[w_base + s], osems[b])

            nxt = s + NBUF_G - 1
            if nxt < seq_per_w:
                nb = nxt % NBUF_G
                if nxt - NBUF_G in pending_o:
                    pending_o.pop(nxt - NBUF_G).wait()
                pending_g[nxt] = fire_gather(nxt, nb)

        for s in sorted(pending_o):
            pending_o[s].wait()

    return _emb_lookup


def kernel(x, table):
    idx = x.reshape(-1).astype(jnp.int32)
    pe = jnp.asarray(_PE_NP)
    z = _build_transpose()(table.T)
    z_rows = z.reshape((Z_ROWS + Z_PAD) * 2, D_MODEL)
    return _build_emb_lookup()(idx, z_rows, pe)


# speedup vs baseline: 1.2155x; 1.2155x over previous
import functools

import numpy as np
import jax
import jax.numpy as jnp
from jax import lax
from jax.experimental import pallas as pl
from jax.experimental.pallas import tpu as pltpu
from jax.experimental.pallas import tpu_sc as plsc

D_MODEL = 64
SEQ_LEN = 200
BATCH = 1024
V_ROWS = 1000000
SCALE = np.float32(np.sqrt(D_MODEL))

FULL_BLOCKS = V_ROWS // 128
TAIL_ROWS = V_ROWS - FULL_BLOCKS * 128
Z_ROWS = FULL_BLOCKS * 64 + TAIL_ROWS // 2
Z_PAD = 64
NBUF_T = 4
NBUF_G = 3

_SPLIT_A = 128
_SPLIT_B = SEQ_LEN - _SPLIT_A


def _positional_encoding(length, depth):
    half = depth / 2
    positions = np.arange(length)[:, np.newaxis]
    depths = np.arange(half)[np.newaxis, :] / half
    angle_rates = 1 / 10000 ** depths
    angle_rads = positions * angle_rates
    pe = np.concatenate([np.sin(angle_rads), np.cos(angle_rads)], axis=-1)
    return pe.astype(np.float32)


_PE_NP = _positional_encoding(SEQ_LEN, D_MODEL)


@functools.cache
def _build_transpose():
    info = plsc.get_sparse_core_info()
    nc, ns = info.num_cores, info.num_subcores
    nw = nc * ns
    base_n = FULL_BLOCKS // nw
    extra = FULL_BLOCKS - base_n * nw
    slots = base_n + 4
    outer = slots // NBUF_T
    mesh = plsc.VectorSubcoreMesh(core_axis_name="c", subcore_axis_name="s")

    @functools.partial(
        pl.kernel,
        mesh=mesh,
        out_type=jax.ShapeDtypeStruct((Z_ROWS + Z_PAD, 128), jnp.float32),
        scratch_types=[
            [pltpu.VMEM((64, 128), jnp.float32)] * NBUF_T,
            [pltpu.VMEM((64, 128), jnp.float32)] * NBUF_T,
            pltpu.VMEM((64, 64), jnp.float32),
            pltpu.VMEM((32, 128), jnp.float32),
            [pltpu.SemaphoreType.DMA] * NBUF_T,
            [pltpu.SemaphoreType.DMA] * NBUF_T,
        ],
        compiler_params=pltpu.CompilerParams(needs_layout_passes=False),
    )
    def _transpose(tbl_t, z, in_bufs, out_bufs, tail_in, tail_out,
                   isems, osems):
        wid = lax.axis_index("s") * nc + lax.axis_index("c")
        n_w = base_n + jnp.where(wid < extra, 1, 0)
        start = base_n * wid + jnp.minimum(wid, extra)

        iota = lax.iota(jnp.int32, 16)
        rows_t = [iota + 16 * t for t in range(4)]
        diag = [(iota + k) & 15 for k in range(16)]
        dshr = [d >> 1 for d in diag]
        colv8 = [[rows_t[cb] + ((diag[k & 1] & 1) << 6) for cb in range(4)]
                 for k in range(2)]

        def src_col(blk):
            g = jnp.where(blk < n_w, start + blk, 0)
            return g * 128

        def dst_row(blk):
            return jnp.where(blk < n_w, (start + blk) * 64, Z_ROWS)

        def fire_in(blk, b):
            pltpu.async_copy(
                tbl_t.at[:, pl.ds(src_col(blk), 128)], in_bufs[b], isems[b])

        def fire_out(blk, b):
            pltpu.async_copy(
                out_bufs[b], z.at[pl.ds(dst_row(blk), 64)], osems[b])

        def transpose_block(src, dst):
            def body(jb, carry):
                j0 = jb * 16
                jhalf = jb * 8
                for k in range(16):
                    srccol = diag[k] + j0
                    rowv = dshr[k] + jhalf
                    for cb in range(4):
                        v = plsc.load_gather(src, [rows_t[cb], srccol])
                        plsc.store_scatter(dst, [rowv, colv8[k & 1][cb]], v)
                return carry
            lax.fori_loop(0, 8, body, 0, unroll=2)

        for b in range(NBUF_T):
            fire_in(b, b)

        def outer_body(k, carry):
            for b in range(NBUF_T):
                blk = k * NBUF_T + b
                pltpu.make_async_copy(
                    tbl_t.at[:, pl.ds(0, 128)], in_bufs[b], isems[b]).wait()

                @pl.when(k > 0)
                def _():
                    pltpu.make_async_copy(
                        out_bufs[b], z.at[pl.ds(0, 64)], osems[b]).wait()

                transpose_block(in_bufs[b], out_bufs[b])
                fire_out(blk, b)

                @pl.when(k < outer - 1)
                def _():
                    fire_in(blk + NBUF_T, b)
            return carry

        lax.fori_loop(0, outer, outer_body, 0)
        for b in range(NBUF_T):
            pltpu.make_async_copy(
                out_bufs[b], z.at[pl.ds(0, 64)], osems[b]).wait()

        @pl.when(wid == nw - 1)
        def _():
            pltpu.sync_copy(tbl_t.at[:, pl.ds(FULL_BLOCKS * 128, TAIL_ROWS)],
                            tail_in)

            def tail_body(p, carry):
                c0 = jnp.full((16,), 2 * p, jnp.int32)
                c1 = c0 + 1
                for t in range(4):
                    v = plsc.load_gather(tail_in, [rows_t[t], c0])
                    tail_out[p, pl.ds(16 * t, 16)] = v
                for t in range(4):
                    v = plsc.load_gather(tail_in, [rows_t[t], c1])
                    tail_out[p, pl.ds(64 + 16 * t, 16)] = v
                return carry
            lax.fori_loop(0, TAIL_ROWS // 2, tail_body, 0)
            pltpu.sync_copy(tail_out,
                            z.at[pl.ds(FULL_BLOCKS * 64, TAIL_ROWS // 2)])

    return _transpose


@functools.cache
def _build_emb_lookup():
    info = plsc.get_sparse_core_info()
    nc, ns = info.num_cores, info.num_subcores
    nw = nc * ns
    seq_per_w = BATCH // nw
    n_idx = seq_per_w * SEQ_LEN
    mesh = plsc.VectorSubcoreMesh(core_axis_name="c", subcore_axis_name="s")

    @functools.partial(
        pl.kernel,
        mesh=mesh,
        out_type=jax.ShapeDtypeStruct((BATCH, SEQ_LEN, D_MODEL), jnp.float32),
        scratch_types=[
            pltpu.VMEM((n_idx,), jnp.int32),
            pltpu.VMEM((SEQ_LEN, D_MODEL), jnp.float32),
            [pltpu.VMEM((SEQ_LEN, D_MODEL), jnp.float32)] * NBUF_G,
            [pltpu.SemaphoreType.DMA] * NBUF_G,
            [pltpu.SemaphoreType.DMA] * NBUF_G,
        ],
        compiler_params=pltpu.CompilerParams(use_tc_tiling_on_sc=False),
    )
    def _emb_lookup(idx_hbm, z_hbm, pe_hbm, out_hbm,
                    idx_v, pe_v, bufs, gsems, osems):
        wid = lax.axis_index("s") * nc + lax.axis_index("c")
        w_base = wid * seq_per_w

        pltpu.sync_copy(pe_hbm, pe_v)
        pltpu.sync_copy(idx_hbm.at[pl.ds(w_base * SEQ_LEN, n_idx)], idx_v)

        def fire_gather(s, b):
            base = s * SEQ_LEN
            c1 = pltpu.async_copy(
                z_hbm.at[idx_v.at[pl.ds(base, _SPLIT_A)]],
                bufs[b].at[pl.ds(0, _SPLIT_A)], gsems[b])
            c2 = pltpu.async_copy(
                z_hbm.at[idx_v.at[pl.ds(base + _SPLIT_A, _SPLIT_B)]],
                bufs[b].at[pl.ds(_SPLIT_A, _SPLIT_B)], gsems[b])
            return c1, c2

        pending_g = {}
        pending_o = {}
        for s in range(NBUF_G - 1):
            pending_g[s] = fire_gather(s, s)

        for s in range(seq_per_w):
            b = s % NBUF_G
            c1, c2 = pending_g.pop(s)
            c1.wait()
            c2.wait()
            buf = bufs[b]

            def row_body(r, c, buf=buf):
                for j in range(D_MODEL // 16):
                    sl = pl.ds(j * 16, 16)
                    buf[r, sl] = buf[r, sl] * SCALE + pe_v[r, sl]
                return c

            lax.fori_loop(0, SEQ_LEN, row_body, 0, unroll=2)

            pending_o[s] = pltpu.async_copy(
                buf, out_hbm.at[w_base + s], osems[b])

            nxt = s + NBUF_G - 1
            if nxt < seq_per_w:
                nb = nxt % NBUF_G
                if nxt - NBUF_G in pending_o:
                    pending_o.pop(nxt - NBUF_G).wait()
                pending_g[nxt] = fire_gather(nxt, nb)

        for s in sorted(pending_o):
            pending_o[s].wait()

    return _emb_lookup


def kernel(x, table):
    idx = x.reshape(-1).astype(jnp.int32)
    pe = jnp.asarray(_PE_NP)
    z = _build_transpose()(table.T)
    z_rows = z.reshape((Z_ROWS + Z_PAD) * 2, D_MODEL)
    return _build_emb_lookup()(idx, z_rows, pe)


# speedup vs baseline: 1.2159x; 1.0003x over previous
import functools

import numpy as np
import jax
import jax.numpy as jnp
from jax import lax
from jax.experimental import pallas as pl
from jax.experimental.pallas import tpu as pltpu
from jax.experimental.pallas import tpu_sc as plsc

D_MODEL = 64
SEQ_LEN = 200
BATCH = 1024
V_ROWS = 1000000
SCALE = np.float32(np.sqrt(D_MODEL))

FULL_BLOCKS = V_ROWS // 128
TAIL_ROWS = V_ROWS - FULL_BLOCKS * 128
Z_ROWS = FULL_BLOCKS * 64 + TAIL_ROWS // 2
Z_PAD = 64
NBUF_T = 4
NBUF_G = 4

_SPLIT_A = 128
_SPLIT_B = SEQ_LEN - _SPLIT_A


def _positional_encoding(length, depth):
    half = depth / 2
    positions = np.arange(length)[:, np.newaxis]
    depths = np.arange(half)[np.newaxis, :] / half
    angle_rates = 1 / 10000 ** depths
    angle_rads = positions * angle_rates
    pe = np.concatenate([np.sin(angle_rads), np.cos(angle_rads)], axis=-1)
    return pe.astype(np.float32)


_PE_NP = _positional_encoding(SEQ_LEN, D_MODEL)


@functools.cache
def _build_transpose():
    info = plsc.get_sparse_core_info()
    nc, ns = info.num_cores, info.num_subcores
    nw = nc * ns
    base_n = FULL_BLOCKS // nw
    extra = FULL_BLOCKS - base_n * nw
    slots = base_n + 4
    outer = slots // NBUF_T
    mesh = plsc.VectorSubcoreMesh(core_axis_name="c", subcore_axis_name="s")

    @functools.partial(
        pl.kernel,
        mesh=mesh,
        out_type=jax.ShapeDtypeStruct((Z_ROWS + Z_PAD, 128), jnp.float32),
        scratch_types=[
            [pltpu.VMEM((64, 128), jnp.float32)] * NBUF_T,
            [pltpu.VMEM((64, 128), jnp.float32)] * NBUF_T,
            pltpu.VMEM((64, 64), jnp.float32),
            pltpu.VMEM((32, 128), jnp.float32),
            [pltpu.SemaphoreType.DMA] * NBUF_T,
            [pltpu.SemaphoreType.DMA] * NBUF_T,
        ],
        compiler_params=pltpu.CompilerParams(needs_layout_passes=False),
    )
    def _transpose(tbl_t, z, in_bufs, out_bufs, tail_in, tail_out,
                   isems, osems):
        wid = lax.axis_index("s") * nc + lax.axis_index("c")
        n_w = base_n + jnp.where(wid < extra, 1, 0)
        start = base_n * wid + jnp.minimum(wid, extra)

        iota = lax.iota(jnp.int32, 16)
        rows_t = [iota + 16 * t for t in range(4)]
        diag = [(iota + k) & 15 for k in range(16)]
        dshr = [d >> 1 for d in diag]
        colv8 = [[rows_t[cb] + ((diag[k & 1] & 1) << 6) for cb in range(4)]
                 for k in range(2)]

        def src_col(blk):
            g = jnp.where(blk < n_w, start + blk, 0)
            return g * 128

        def dst_row(blk):
            return jnp.where(blk < n_w, (start + blk) * 64, Z_ROWS)

        def fire_in(blk, b):
            pltpu.async_copy(
                tbl_t.at[:, pl.ds(src_col(blk), 128)], in_bufs[b], isems[b])

        def fire_out(blk, b):
            pltpu.async_copy(
                out_bufs[b], z.at[pl.ds(dst_row(blk), 64)], osems[b])

        def transpose_block(src, dst):
            def body(jb, carry):
                j0 = jb * 16
                jhalf = jb * 8
                for k in range(16):
                    srccol = diag[k] + j0
                    rowv = dshr[k] + jhalf
                    for cb in range(4):
                        v = plsc.load_gather(src, [rows_t[cb], srccol])
                        plsc.store_scatter(dst, [rowv, colv8[k & 1][cb]], v)
                return carry
            lax.fori_loop(0, 8, body, 0, unroll=2)

        for b in range(NBUF_T):
            fire_in(b, b)

        def outer_body(k, carry):
            for b in range(NBUF_T):
                blk = k * NBUF_T + b
                pltpu.make_async_copy(
                    tbl_t.at[:, pl.ds(0, 128)], in_bufs[b], isems[b]).wait()

                @pl.when(k > 0)
                def _():
                    pltpu.make_async_copy(
                        out_bufs[b], z.at[pl.ds(0, 64)], osems[b]).wait()

                transpose_block(in_bufs[b], out_bufs[b])
                fire_out(blk, b)

                @pl.when(k < outer - 1)
                def _():
                    fire_in(blk + NBUF_T, b)
            return carry

        lax.fori_loop(0, outer, outer_body, 0)
        for b in range(NBUF_T):
            pltpu.make_async_copy(
                out_bufs[b], z.at[pl.ds(0, 64)], osems[b]).wait()

        @pl.when(wid == nw - 1)
        def _():
            pltpu.sync_copy(tbl_t.at[:, pl.ds(FULL_BLOCKS * 128, TAIL_ROWS)],
                            tail_in)

            def tail_body(p, carry):
                c0 = jnp.full((16,), 2 * p, jnp.int32)
                c1 = c0 + 1
                for t in range(4):
                    v = plsc.load_gather(tail_in, [rows_t[t], c0])
                    tail_out[p, pl.ds(16 * t, 16)] = v
                for t in range(4):
                    v = plsc.load_gather(tail_in, [rows_t[t], c1])
                    tail_out[p, pl.ds(64 + 16 * t, 16)] = v
                return carry
            lax.fori_loop(0, TAIL_ROWS // 2, tail_body, 0)
            pltpu.sync_copy(tail_out,
                            z.at[pl.ds(FULL_BLOCKS * 64, TAIL_ROWS // 2)])

    return _transpose


@functools.cache
def _build_emb_lookup():
    info = plsc.get_sparse_core_info()
    nc, ns = info.num_cores, info.num_subcores
    nw = nc * ns
    seq_per_w = BATCH // nw
    n_idx = seq_per_w * SEQ_LEN
    mesh = plsc.VectorSubcoreMesh(core_axis_name="c", subcore_axis_name="s")

    @functools.partial(
        pl.kernel,
        mesh=mesh,
        out_type=jax.ShapeDtypeStruct((BATCH, SEQ_LEN, D_MODEL), jnp.float32),
        scratch_types=[
            pltpu.VMEM((n_idx,), jnp.int32),
            pltpu.VMEM((SEQ_LEN, D_MODEL), jnp.float32),
            [pltpu.VMEM((SEQ_LEN, D_MODEL), jnp.float32)] * NBUF_G,
            [pltpu.SemaphoreType.DMA] * NBUF_G,
            [pltpu.SemaphoreType.DMA] * NBUF_G,
        ],
        compiler_params=pltpu.CompilerParams(use_tc_tiling_on_sc=False),
    )
    def _emb_lookup(idx_hbm, z_hbm, pe_hbm, out_hbm,
                    idx_v, pe_v, bufs, gsems, osems):
        wid = lax.axis_index("s") * nc + lax.axis_index("c")
        w_base = wid * seq_per_w

        pltpu.sync_copy(pe_hbm, pe_v)
        pltpu.sync_copy(idx_hbm.at[pl.ds(w_base * SEQ_LEN, n_idx)], idx_v)

        def fire_gather(s, b):
            base = s * SEQ_LEN
            c1 = pltpu.async_copy(
                z_hbm.at[idx_v.at[pl.ds(base, _SPLIT_A)]],
                bufs[b].at[pl.ds(0, _SPLIT_A)], gsems[b])
            c2 = pltpu.async_copy(
                z_hbm.at[idx_v.at[pl.ds(base + _SPLIT_A, _SPLIT_B)]],
                bufs[b].at[pl.ds(_SPLIT_A, _SPLIT_B)], gsems[b])
            return c1, c2

        pending_g = {}
        pending_o = {}
        for s in range(NBUF_G - 1):
            pending_g[s] = fire_gather(s, s)

        for s in range(seq_per_w):
            b = s % NBUF_G
            c1, c2 = pending_g.pop(s)
            c1.wait()
            c2.wait()
            buf = bufs[b]

            def row_body(r, c, buf=buf):
                for j in range(D_MODEL // 16):
                    sl = pl.ds(j * 16, 16)
                    buf[r, sl] = buf[r, sl] * SCALE + pe_v[r, sl]
                return c

            lax.fori_loop(0, SEQ_LEN, row_body, 0, unroll=2)

            pending_o[s] = pltpu.async_copy(
                buf, out_hbm.at[w_base + s], osems[b])

            nxt = s + NBUF_G - 1
            if nxt < seq_per_w:
                nb = nxt % NBUF_G
                if nxt - NBUF_G in pending_o:
                    pending_o.pop(nxt - NBUF_G).wait()
                pending_g[nxt] = fire_gather(nxt, nb)

        for s in sorted(pending_o):
            pending_o[s].wait()

    return _emb_lookup


def kernel(x, table):
    idx = x.reshape(-1).astype(jnp.int32)
    pe = jnp.asarray(_PE_NP)
    z = _build_transpose()(table.T)
    z_rows = z.reshape((Z_ROWS + Z_PAD) * 2, D_MODEL)
    return _build_emb_lookup()(idx, z_rows, pe)
